# baseline (speedup 1.0000x reference)
import functools

import jax
import jax.numpy as jnp
from jax import lax
from jax.experimental import pallas as pl
from jax.experimental.pallas import tpu as pltpu
from jax.experimental.pallas import tpu_sc as plsc

EMBED = 32
NC, NS = 2, 16
NW = NC * NS
BW = 128
K = 2
L = 16


@jax.jit
def _gather_sc(xT, table):
    length, batch = xT.shape
    assert batch == NW * BW
    n_iters = length // K
    assert length % K == 0
    mesh = plsc.VectorSubcoreMesh(core_axis_name="c", subcore_axis_name="s")

    @functools.partial(
        pl.kernel,
        out_type=jax.ShapeDtypeStruct(
            (length, EMBED // 8, NW, 8, BW), jnp.float32),
        mesh=mesh,
        compiler_params=pltpu.CompilerParams(
            use_tc_tiling_on_sc=False, needs_layout_passes=False),
        scratch_types=(
            [pltpu.VMEM((length, BW), jnp.int32)]
            + [pltpu.VMEM((BW, EMBED), jnp.float32) for _ in range(K)]
            + [pltpu.VMEM((EMBED // 8, 8, BW), jnp.float32) for _ in range(K)]
            + [pltpu.SemaphoreType.DMA for _ in range(2 * K)]
        ),
    )
    def k(xT_hbm, table_hbm, out_hbm, idx_v, *bufs_sems):
        bufs = bufs_sems[:K]
        tbufs = bufs_sems[K:2 * K]
        gsem = bufs_sems[2 * K:3 * K]
        wsem = bufs_sems[3 * K:]
        wid = lax.axis_index("s") * NC + lax.axis_index("c")
        b0 = wid * BW
        pltpu.sync_copy(xT_hbm.at[:, pl.ds(b0, BW)], idx_v)

        lanes = lax.iota(jnp.int32, L)

        def fire_gather(b, t):
            return pltpu.async_copy(
                table_hbm.at[idx_v.at[t]], bufs[b], gsem[b])

        for b in range(K):
            fire_gather(b, b)

        def body(i, carry):
            for b in range(K):
                t = i * K + b
                pltpu.make_async_copy(
                    table_hbm.at[idx_v.at[0]], bufs[b], gsem[b]).wait()
                def drain_tbuf(b=b):
                    pltpu.make_async_copy(
                        tbufs[b], out_hbm.at[0, :, wid], wsem[b]).wait()
                pl.when(i != 0)(drain_tbuf)
                for e in range(EMBED):
                    col = jnp.full((L,), e, jnp.int32)
                    for v0 in range(0, BW, L):
                        rows = lanes + v0
                        vec = plsc.load_gather(bufs[b], [rows, col])
                        tbufs[b][e // 8, e % 8, pl.ds(v0, L)] = vec
                pltpu.async_copy(
                    tbufs[b], out_hbm.at[t, :, wid], wsem[b])

                @pl.when(t + K < length)
                def _():
                    fire_gather(b, t + K)
            return carry

        lax.fori_loop(0, n_iters, body, 0)
        for b in range(K):
            pltpu.make_async_copy(
                tbufs[b], out_hbm.at[0, :, wid], wsem[b]).wait()

    return k(xT, table)


def kernel(x, idx2vec):
    batch, length = x.shape
    out5 = _gather_sc(x.T, idx2vec)
    return out5.transpose(2, 4, 0, 1, 3).reshape(batch, length, EMBED)


# speedup vs baseline: 1.6947x; 1.6947x over previous
import functools

import jax
import jax.numpy as jnp
from jax import lax
from jax.experimental import pallas as pl
from jax.experimental.pallas import tpu as pltpu
from jax.experimental.pallas import tpu_sc as plsc

EMBED = 32
NC, NS = 2, 16
NW = NC * NS
BW = 128
BWP = BW + 1
K = 2
L = 16


@jax.jit
def _gather_sc(xT, table):
    length, batch = xT.shape
    assert batch == NW * BW
    n_iters = length // K
    assert length % K == 0
    mesh = plsc.VectorSubcoreMesh(core_axis_name="c", subcore_axis_name="s")

    @functools.partial(
        pl.kernel,
        out_type=jax.ShapeDtypeStruct(
            (length, EMBED // 8, NW, 8, BW), jnp.float32),
        mesh=mesh,
        compiler_params=pltpu.CompilerParams(
            use_tc_tiling_on_sc=False, needs_layout_passes=False),
        scratch_types=(
            [pltpu.VMEM((length, BW), jnp.int32)]
            + [pltpu.VMEM((BW, EMBED), jnp.float32) for _ in range(K)]
            + [pltpu.VMEM((EMBED // 8, 8, BWP), jnp.float32) for _ in range(K)]
            + [pltpu.SemaphoreType.DMA for _ in range(2 * K)]
        ),
    )
    def k(xT_hbm, table_hbm, out_hbm, idx_v, *bufs_sems):
        bufs = bufs_sems[:K]
        tbufs = bufs_sems[K:2 * K]
        gsem = bufs_sems[2 * K:3 * K]
        wsem = bufs_sems[3 * K:]
        wid = lax.axis_index("s") * NC + lax.axis_index("c")
        b0 = wid * BW
        pltpu.sync_copy(xT_hbm.at[:, pl.ds(b0, BW)], idx_v)

        lanes = lax.iota(jnp.int32, L)
        ebs = [(lanes + e0) // 8 for e0 in (0, L)]
        ess = [(lanes + e0) % 8 for e0 in (0, L)]

        def tbuf_window(b):
            return tbufs[b].at[:, :, pl.ds(0, BW)]

        def fire_gather(b, t):
            return pltpu.async_copy(
                table_hbm.at[idx_v.at[t]], bufs[b], gsem[b])

        for b in range(K):
            fire_gather(b, b)

        def body(i, carry):
            for b in range(K):
                t = i * K + b
                pltpu.make_async_copy(
                    table_hbm.at[idx_v.at[0]], bufs[b], gsem[b]).wait()

                def drain_tbuf(b=b):
                    pltpu.make_async_copy(
                        tbuf_window(b), out_hbm.at[0, :, wid], wsem[b]).wait()
                pl.when(i != 0)(drain_tbuf)
                for v in range(BW):
                    vcol = jnp.full((L,), v, jnp.int32)
                    for h in range(2):
                        vec = bufs[b][v, pl.ds(h * L, L)]
                        plsc.store_scatter(
                            tbufs[b], [ebs[h], ess[h], vcol], vec)
                pltpu.async_copy(
                    tbuf_window(b), out_hbm.at[t, :, wid], wsem[b])

                @pl.when(t + K < length)
                def _():
                    fire_gather(b, t + K)
            return carry

        lax.fori_loop(0, n_iters, body, 0)
        for b in range(K):
            pltpu.make_async_copy(
                tbuf_window(b), out_hbm.at[0, :, wid], wsem[b]).wait()

    return k(xT, table)


def kernel(x, idx2vec):
    batch, length = x.shape
    out5 = _gather_sc(x.T, idx2vec)
    return out5.transpose(2, 4, 0, 1, 3).reshape(batch, length, EMBED)
